# Initial kernel scaffold; baseline (speedup 1.0000x reference)
import jax
import jax.numpy as jnp
from jax.experimental import pallas as pl


def kernel(feat_text, feat_video, x_hie_conv, x_hie_speaker, edge_index, speaker_index, global_index, text_ln_g, text_ln_b, text_W, text_b, video_ln_g, video_ln_b, video_W, video_b, tok_hc, tok_sp, g1_Wl, g1_bl, g1_Wr, g1_br, g1_att, g1_bias, g2_Wl, g2_bl, g2_Wr, g2_br, g2_att, g2_bias, g3_Wl, g3_bl, g3_Wr, g3_br, g3_att, g3_bias, dec_W1, dec_b1, dec_ln_g, dec_ln_b, dec_W2, dec_b2):
    raise NotImplementedError("write your pallas kernel here")


# speedup vs baseline: 1.0014x; 1.0014x over previous
import functools

import jax
import jax.numpy as jnp
from jax.experimental import pallas as pl
from jax.experimental.pallas import tpu as pltpu

H = 4
HID = 128


def _lrelu(x, s=0.2):
    return jnp.where(x >= 0, x, s * x)


def _enc_body(ft_ref, fv_ref, tg_ref, tb_ref, tw_ref, tbias_ref,
              vg_ref, vb_ref, vw_ref, vbias_ref, out_ref):
    ft = ft_ref[...]
    fv = fv_ref[...]
    m = jnp.mean(ft, axis=-1, keepdims=True)
    v = jnp.mean((ft - m) ** 2, axis=-1, keepdims=True)
    nt = (ft - m) * jax.lax.rsqrt(v + 1e-5) * tg_ref[...] + tb_ref[...]
    m2 = jnp.mean(fv, axis=-1, keepdims=True)
    v2 = jnp.mean((fv - m2) ** 2, axis=-1, keepdims=True)
    nv = (fv - m2) * jax.lax.rsqrt(v2 + 1e-5) * vg_ref[...] + vb_ref[...]
    pt = jax.nn.relu(jax.lax.dot_general(nt, tw_ref[...],
                                         (((1,), (1,)), ((), ())),
                                         preferred_element_type=jnp.float32)
                     + tbias_ref[...])
    pv = jax.nn.relu(jax.lax.dot_general(nv, vw_ref[...],
                                         (((1,), (1,)), ((), ())),
                                         preferred_element_type=jnp.float32)
                     + vbias_ref[...])
    out_ref[...] = (pt + pv) * 0.5


def _encoder(ft, fv, tg, tb, tw, tbias, vg, vb, vw, vbias):
    n = ft.shape[0]
    blk = 1000
    grid = (n // blk,)
    return pl.pallas_call(
        _enc_body,
        grid=grid,
        in_specs=[
            pl.BlockSpec((blk, ft.shape[1]), lambda i: (i, 0)),
            pl.BlockSpec((blk, fv.shape[1]), lambda i: (i, 0)),
            pl.BlockSpec((ft.shape[1],), lambda i: (0,)),
            pl.BlockSpec((ft.shape[1],), lambda i: (0,)),
            pl.BlockSpec((HID, ft.shape[1]), lambda i: (0, 0)),
            pl.BlockSpec((HID,), lambda i: (0,)),
            pl.BlockSpec((fv.shape[1],), lambda i: (0,)),
            pl.BlockSpec((fv.shape[1],), lambda i: (0,)),
            pl.BlockSpec((HID, fv.shape[1]), lambda i: (0, 0)),
            pl.BlockSpec((HID,), lambda i: (0,)),
        ],
        out_specs=pl.BlockSpec((blk, HID), lambda i: (i, 0)),
        out_shape=jax.ShapeDtypeStruct((n, HID), jnp.float32),
    )(ft, fv, tg, tb, tw, tbias, vg, vb, vw, vbias)


def _gatv2(x, src, dst, Wl, bl, Wr, br, att, bias):
    N = x.shape[0]
    C = att.shape[-1]
    xl = (x @ Wl.T + bl).reshape(N, H, C)
    xr = (x @ Wr.T + br).reshape(N, H, C)
    m = _lrelu(xl[src] + xr[dst])
    e = jnp.sum(m * att[None, :, :], axis=-1)
    emax = jax.ops.segment_max(e, dst, num_segments=N)
    emax = jnp.where(jnp.isfinite(emax), emax, 0.0)
    ee = jnp.exp(e - emax[dst])
    den = jax.ops.segment_sum(ee, dst, num_segments=N)
    alpha = ee / (den[dst] + 1e-16)
    out = jax.ops.segment_sum(alpha[:, :, None] * xl[src], dst, num_segments=N)
    return out.reshape(N, H * C) + bias


def kernel(feat_text, feat_video, x_hie_conv, x_hie_speaker, edge_index, speaker_index, global_index, text_ln_g, text_ln_b, text_W, text_b, video_ln_g, video_ln_b, video_W, video_b, tok_hc, tok_sp, g1_Wl, g1_bl, g1_Wr, g1_br, g1_att, g1_bias, g2_Wl, g2_bl, g2_Wr, g2_br, g2_att, g2_bias, g3_Wl, g3_bl, g3_Wr, g3_br, g3_att, g3_bias, dec_W1, dec_b1, dec_ln_g, dec_ln_b, dec_W2, dec_b2):
    x = _encoder(feat_text, feat_video, text_ln_g, text_ln_b, text_W, text_b,
                 video_ln_g, video_ln_b, video_W, video_b)
    hc = jnp.tile(tok_hc[None, :], (x_hie_conv.shape[0], 1))
    sp = jnp.tile(tok_sp[None, :], (x_hie_speaker.shape[0], 1))
    h = jnp.concatenate([x, hc, sp], axis=0)
    N = h.shape[0]
    loops = jnp.arange(N, dtype=edge_index.dtype)
    src = jnp.concatenate([edge_index[0], loops])
    dst = jnp.concatenate([edge_index[1], loops])
    h = _gatv2(h, src, dst, g1_Wl, g1_bl, g1_Wr, g1_br, g1_att, g1_bias)
    h = _gatv2(h, src, dst, g2_Wl, g2_bl, g2_Wr, g2_br, g2_att, g2_bias)
    h = _gatv2(h, src, dst, g3_Wl, g3_bl, g3_Wr, g3_br, g3_att, g3_bias)
    glob = jax.lax.dynamic_slice_in_dim(h, global_index, 1, axis=0)
    z1 = glob @ dec_W1.T + dec_b1
    mz = jnp.mean(z1, axis=-1, keepdims=True)
    vz = jnp.mean((z1 - mz) ** 2, axis=-1, keepdims=True)
    z1 = jax.nn.relu((z1 - mz) / jnp.sqrt(vz + 1e-5) * dec_ln_g + dec_ln_b)
    z = z1 @ dec_W2.T + dec_b2
    return z


# speedup vs baseline: 10.8372x; 10.8217x over previous
import functools

import jax
import jax.numpy as jnp
from jax import lax
from jax.experimental import pallas as pl
from jax.experimental.pallas import tpu as pltpu
from jax.experimental.pallas import tpu_sc as plsc

H = 4
HID = 128
N_CONV = 10000
N_TOT = 10600
NP = 10624
ROWS_PER_TILE = NP // 16
E_RAW = 169600 + N_TOT
EP = 180224
E_PER_TILE = EP // 32
K = 64
SUP = 8
CHUNKS = E_PER_TILE // K
SUPS = CHUNKS // SUP
EROWS = EP // K
NPQ = 1408
DROWS_PER_TILE = NPQ // 16


def _enc_body(ft_ref, fv_ref, tg_ref, tb_ref, tw_ref, tbias_ref,
              vg_ref, vb_ref, vw_ref, vbias_ref, out_ref):
    ft = ft_ref[...]
    fv = fv_ref[...]
    m = jnp.mean(ft, axis=-1, keepdims=True)
    v = jnp.mean((ft - m) ** 2, axis=-1, keepdims=True)
    nt = (ft - m) * lax.rsqrt(v + 1e-5) * tg_ref[...] + tb_ref[...]
    m2 = jnp.mean(fv, axis=-1, keepdims=True)
    v2 = jnp.mean((fv - m2) ** 2, axis=-1, keepdims=True)
    nv = (fv - m2) * lax.rsqrt(v2 + 1e-5) * vg_ref[...] + vb_ref[...]
    pt = jax.nn.relu(lax.dot_general(nt, tw_ref[...], (((1,), (1,)), ((), ())),
                                     preferred_element_type=jnp.float32)
                     + tbias_ref[...])
    pv = jax.nn.relu(lax.dot_general(nv, vw_ref[...], (((1,), (1,)), ((), ())),
                                     preferred_element_type=jnp.float32)
                     + vbias_ref[...])
    out_ref[...] = (pt + pv) * 0.5


def _encoder(ft, fv, tg, tb, tw, tbias, vg, vb, vw, vbias):
    n = ft.shape[0]
    blk = 1000
    return pl.pallas_call(
        _enc_body,
        grid=(n // blk,),
        in_specs=[
            pl.BlockSpec((blk, ft.shape[1]), lambda i: (i, 0)),
            pl.BlockSpec((blk, fv.shape[1]), lambda i: (i, 0)),
            pl.BlockSpec((ft.shape[1],), lambda i: (0,)),
            pl.BlockSpec((ft.shape[1],), lambda i: (0,)),
            pl.BlockSpec((HID, ft.shape[1]), lambda i: (0, 0)),
            pl.BlockSpec((HID,), lambda i: (0,)),
            pl.BlockSpec((fv.shape[1],), lambda i: (0,)),
            pl.BlockSpec((fv.shape[1],), lambda i: (0,)),
            pl.BlockSpec((HID, fv.shape[1]), lambda i: (0, 0)),
            pl.BlockSpec((HID,), lambda i: (0,)),
        ],
        out_specs=pl.BlockSpec((blk, HID), lambda i: (i, 0)),
        out_shape=jax.ShapeDtypeStruct((n, HID), jnp.float32),
    )(ft, fv, tg, tb, tw, tbias, vg, vb, vw, vbias)


def _proj_body(h_ref, wl_ref, bl_ref, wr_ref, br_ref, xl_ref, xr_ref):
    hb = h_ref[...]
    for hd in range(H):
        wl = wl_ref[pl.ds(hd * HID, HID), :]
        wr = wr_ref[pl.ds(hd * HID, HID), :]
        xl_ref[hd] = lax.dot_general(hb, wl, (((1,), (1,)), ((), ())),
                                     preferred_element_type=jnp.float32) \
            + bl_ref[pl.ds(hd * HID, HID)]
        xr_ref[hd] = lax.dot_general(hb, wr, (((1,), (1,)), ((), ())),
                                     preferred_element_type=jnp.float32) \
            + br_ref[pl.ds(hd * HID, HID)]


def _project(h, Wl, bl, Wr, br):
    din = h.shape[1]
    blk = ROWS_PER_TILE
    out = jax.ShapeDtypeStruct((H, NP, HID), jnp.float32)
    return pl.pallas_call(
        _proj_body,
        grid=(NP // blk,),
        in_specs=[
            pl.BlockSpec((blk, din), lambda i: (i, 0)),
            pl.BlockSpec((H * HID, din), lambda i: (0, 0)),
            pl.BlockSpec((H * HID,), lambda i: (0,)),
            pl.BlockSpec((H * HID, din), lambda i: (0, 0)),
            pl.BlockSpec((H * HID,), lambda i: (0,)),
        ],
        out_specs=[
            pl.BlockSpec((H, blk, HID), lambda i: (0, i, 0)),
            pl.BlockSpec((H, blk, HID), lambda i: (0, i, 0)),
        ],
        out_shape=[out, out],
    )(h, Wl, bl, Wr, br)


def _combine_body(num_ref, den_ref, bias_ref, out_ref):
    for hd in range(H):
        n = num_ref[0, hd] + num_ref[1, hd]
        d = den_ref[0, hd, :, 0:1] + den_ref[1, hd, :, 0:1]
        out_ref[:, pl.ds(hd * HID, HID)] = (
            n / (d + 1e-16) + bias_ref[pl.ds(hd * HID, HID)])


def _combine(num, den16, bias):
    blk = ROWS_PER_TILE
    return pl.pallas_call(
        _combine_body,
        grid=(NP // blk,),
        in_specs=[
            pl.BlockSpec((2, H, blk, HID), lambda i: (0, 0, i, 0)),
            pl.BlockSpec((2, H, blk, 16), lambda i: (0, 0, i, 0)),
            pl.BlockSpec((H * HID,), lambda i: (0,)),
        ],
        out_specs=pl.BlockSpec((blk, H * HID), lambda i: (i, 0)),
        out_shape=jax.ShapeDtypeStruct((NP, H * HID), jnp.float32),
    )(num, den16, bias)


def _dec_body(g_ref, w1_ref, b1_ref, lg_ref, lb_ref, w2_ref, b2_ref, out_ref):
    z1 = lax.dot_general(g_ref[...], w1_ref[...], (((1,), (1,)), ((), ())),
                         preferred_element_type=jnp.float32) + b1_ref[...]
    m = jnp.mean(z1, axis=-1, keepdims=True)
    v = jnp.mean((z1 - m) ** 2, axis=-1, keepdims=True)
    z1 = jax.nn.relu((z1 - m) * lax.rsqrt(v + 1e-5) * lg_ref[...] + lb_ref[...])
    out_ref[...] = jnp.sum(z1 * w2_ref[...], axis=-1, keepdims=True) \
        + b2_ref[...]


def _decoder(glob, W1, b1, lg, lb, W2, b2):
    return pl.pallas_call(
        _dec_body,
        out_shape=jax.ShapeDtypeStruct((1, 1), jnp.float32),
    )(glob, W1, b1, lg, lb, W2, b2.reshape(1, 1))


def _edge_body(xl_ref, xr_ref, att_ref, srcg_ref, dstg_ref, dsts_ref,
               dstq_ref,
               num_out, den_out,
               num_s, den_s,
               A, B, D2, ig, jg, sc, dq, scv, scv2,
               attv):
    c = lax.axis_index("c")
    s = lax.axis_index("s")
    wid = c * 16 + s
    rows0 = s * ROWS_PER_TILE
    rows0d = s * DROWS_PER_TILE

    pltpu.sync_copy(att_ref, attv)

    zv = jnp.zeros((16,), jnp.float32)

    lanes = lax.broadcasted_iota(jnp.int32, (16,), 0)
    lane0 = jnp.where(lanes == 0, 1.0, 0.0)

    base_r = wid * CHUNKS

    def zrow(i, carry):
        for j in range(HID // 16):
            A[i, pl.ds(j * 16, 16)] = zv
            D2[i, pl.ds(j * 16, 16)] = zv
        return carry

    nz = (ROWS_PER_TILE + K - 1) // K
    zbases = [min(kk * K, ROWS_PER_TILE - K) for kk in range(nz)]
    nzd = (DROWS_PER_TILE + K - 1) // K
    zbases_d = [min(kk * K, DROWS_PER_TILE - K) for kk in range(nzd)]

    def _set_idx(ref, base):
        for q in range(K // 16):
            ref[pl.ds(q * 16, 16)] = lanes + (base + q * 16)

    for h in range(H):
        lax.fori_loop(0, K, zrow, 0)
        for zb in zbases:
            _set_idx(scv, rows0 + zb)
            pltpu.sync_copy(A, num_s.at[scv])
        for zb in zbases_d:
            _set_idx(scv2, rows0d + zb)
            pltpu.sync_copy(D2, den_s.at[scv2])
        plsc.subcore_barrier()

        def sup_body(si, carry):
            r0 = base_r + si * SUP
            pltpu.sync_copy(srcg_ref.at[h, pl.ds(r0, SUP)], ig)
            pltpu.sync_copy(dstg_ref.at[h, pl.ds(r0, SUP)], jg)
            pltpu.sync_copy(dsts_ref.at[pl.ds(r0, SUP)], sc)
            pltpu.sync_copy(dstq_ref.at[pl.ds(r0, SUP)], dq)
            for b in range(SUP):
                pltpu.sync_copy(xl_ref.at[ig.at[b]], A)
                pltpu.sync_copy(xr_ref.at[jg.at[b]], B)
                for q in range(K // 16):
                    scv[pl.ds(q * 16, 16)] = sc[b, pl.ds(q * 16, 16)]
                    scv2[pl.ds(q * 16, 16)] = dq[b, pl.ds(q * 16, 16)]

                def edge(e, ecarry):
                    acc = zv
                    for j in range(HID // 16):
                        a = A[e, pl.ds(j * 16, 16)]
                        r = B[e, pl.ds(j * 16, 16)]
                        t = a + r
                        m = jnp.maximum(t, t * 0.2)
                        acc = acc + m * attv[pl.ds(h * HID + j * 16, 16)]
                    ev = jnp.exp(jnp.broadcast_to(jnp.sum(acc), (16,)))
                    for j in range(HID // 16):
                        A[e, pl.ds(j * 16, 16)] = A[e, pl.ds(j * 16, 16)] * ev
                    dvec = plsc.load_gather(
                        scv, [jnp.broadcast_to(e, (16,))])
                    t_e = jnp.bitwise_and(dvec, 7)
                    sl0 = ev * lane0
                    for t in range(8):
                        D2[e, pl.ds(t * 16, 16)] = jnp.where(t_e == t, sl0, zv)
                    return ecarry

                lax.fori_loop(0, K, edge, 0)
                pltpu.sync_copy(A, num_s.at[scv], add=True)
                pltpu.sync_copy(D2, den_s.at[scv2], add=True)
            return carry

        lax.fori_loop(0, SUPS, sup_body, 0)
        plsc.subcore_barrier()
        for zb in zbases:
            _set_idx(scv, rows0 + zb)
            pltpu.sync_copy(num_s.at[scv], A)
            pltpu.sync_copy(A, num_out.at[c, h, pl.ds(rows0 + zb, K)])
        for zb in zbases_d:
            _set_idx(scv2, rows0d + zb)
            pltpu.sync_copy(den_s.at[scv2], D2)
            pltpu.sync_copy(D2, den_out.at[c, h, pl.ds(rows0d + zb, K)])
        plsc.subcore_barrier()


@functools.partial(
    pl.kernel,
    out_type=(jax.ShapeDtypeStruct((2, H, NP, HID), jnp.float32),
              jax.ShapeDtypeStruct((2, H, NPQ, HID), jnp.float32)),
    mesh=plsc.VectorSubcoreMesh(core_axis_name="c", subcore_axis_name="s"),
    compiler_params=pltpu.CompilerParams(needs_layout_passes=False),
    scratch_types=[
        pltpu.VMEM_SHARED((NP, HID), jnp.float32),
        pltpu.VMEM_SHARED((NPQ, HID), jnp.float32),
        pltpu.VMEM((K, HID), jnp.float32),
        pltpu.VMEM((K, HID), jnp.float32),
        pltpu.VMEM((K, HID), jnp.float32),
        pltpu.VMEM((SUP, K), jnp.int32),
        pltpu.VMEM((SUP, K), jnp.int32),
        pltpu.VMEM((SUP, K), jnp.int32),
        pltpu.VMEM((SUP, K), jnp.int32),
        pltpu.VMEM((K,), jnp.int32),
        pltpu.VMEM((K,), jnp.int32),
        pltpu.VMEM((H * HID,), jnp.float32),
    ],
)
def _edge_stage(xl_ref, xr_ref, att_ref, srcg_ref, dstg_ref, dsts_ref,
                dstq_ref, num_out, den_out, *scratch):
    _edge_body(xl_ref, xr_ref, att_ref, srcg_ref, dstg_ref, dsts_ref,
               dstq_ref, num_out, den_out, *scratch)


def _gat_layer(h, src_g, dst_g, dst_s, dst_q, Wl, bl, Wr, br, att, bias):
    xl, xr = _project(h, Wl, bl, Wr, br)
    num, den = _edge_stage(xl.reshape(H * NP, HID), xr.reshape(H * NP, HID),
                           att.reshape(H * HID), src_g, dst_g, dst_s, dst_q)
    den_nodes = den.reshape(2, H, NPQ, 8, 16)[..., 0].reshape(
        2, H, NPQ * 8)[:, :, :NP]
    den16 = jnp.broadcast_to(den_nodes[..., None], (2, H, NP, 16))
    return _combine(num, den16, bias)


def kernel(feat_text, feat_video, x_hie_conv, x_hie_speaker, edge_index, speaker_index, global_index, text_ln_g, text_ln_b, text_W, text_b, video_ln_g, video_ln_b, video_W, video_b, tok_hc, tok_sp, g1_Wl, g1_bl, g1_Wr, g1_br, g1_att, g1_bias, g2_Wl, g2_bl, g2_Wr, g2_br, g2_att, g2_bias, g3_Wl, g3_bl, g3_Wr, g3_br, g3_att, g3_bias, dec_W1, dec_b1, dec_ln_g, dec_ln_b, dec_W2, dec_b2):
    x = _encoder(feat_text, feat_video, text_ln_g, text_ln_b, text_W, text_b,
                 video_ln_g, video_ln_b, video_W, video_b)
    hc = jnp.tile(tok_hc[None, :], (x_hie_conv.shape[0], 1))
    sp = jnp.tile(tok_sp[None, :], (x_hie_speaker.shape[0], 1))
    h = jnp.concatenate(
        [x, hc, sp, jnp.zeros((NP - N_TOT, HID), jnp.float32)], axis=0)

    loops = jnp.arange(N_TOT, dtype=jnp.int32)
    pad = EP - E_RAW
    src = jnp.concatenate([edge_index[0].astype(jnp.int32), loops,
                           jnp.zeros((pad,), jnp.int32)])
    dst = jnp.concatenate([edge_index[1].astype(jnp.int32), loops,
                           jnp.full((pad,), N_TOT, jnp.int32)])
    order = jnp.argsort(dst)
    src = src[order].reshape(K, EROWS).T.reshape(EP)
    dst = dst[order].reshape(K, EROWS).T.reshape(EP)

    offs = (jnp.arange(H, dtype=jnp.int32) * NP)[:, None]
    src_g = (src[None, :] + offs).reshape(H, EROWS, K)
    dst_g = (dst[None, :] + offs).reshape(H, EROWS, K)
    dst_q = (dst >> 3).reshape(EROWS, K)
    dst = dst.reshape(EROWS, K)

    h = _gat_layer(h, src_g, dst_g, dst, dst_q, g1_Wl, g1_bl, g1_Wr, g1_br,
                   g1_att, g1_bias)
    h = _gat_layer(h, src_g, dst_g, dst, dst_q, g2_Wl, g2_bl, g2_Wr, g2_br,
                   g2_att, g2_bias)
    h = _gat_layer(h, src_g, dst_g, dst, dst_q, g3_Wl, g3_bl, g3_Wr, g3_br,
                   g3_att, g3_bias)

    glob = lax.dynamic_slice(h, (global_index, 0), (1, H * HID))
    return _decoder(glob, dec_W1, dec_b1, dec_ln_g, dec_ln_b, dec_W2, dec_b2)


# speedup vs baseline: 12.2817x; 1.1333x over previous
import functools

import jax
import jax.numpy as jnp
from jax import lax
from jax.experimental import pallas as pl
from jax.experimental.pallas import tpu as pltpu
from jax.experimental.pallas import tpu_sc as plsc

H = 4
HID = 128
N_CONV = 10000
N_TOT = 10600
NP = 10624
ROWS_PER_TILE = NP // 16
E_RAW = 169600 + N_TOT
EP = 180224
E_PER_TILE = EP // 32
K = 64
SUP = 8
CHUNKS = E_PER_TILE // K
SUPS = CHUNKS // SUP
EROWS = EP // K
NPQ = 1408
DROWS_PER_TILE = NPQ // 16


def _enc_body(ft_ref, fv_ref, tg_ref, tb_ref, tw_ref, tbias_ref,
              vg_ref, vb_ref, vw_ref, vbias_ref, out_ref):
    ft = ft_ref[...]
    fv = fv_ref[...]
    m = jnp.mean(ft, axis=-1, keepdims=True)
    v = jnp.mean((ft - m) ** 2, axis=-1, keepdims=True)
    nt = (ft - m) * lax.rsqrt(v + 1e-5) * tg_ref[...] + tb_ref[...]
    m2 = jnp.mean(fv, axis=-1, keepdims=True)
    v2 = jnp.mean((fv - m2) ** 2, axis=-1, keepdims=True)
    nv = (fv - m2) * lax.rsqrt(v2 + 1e-5) * vg_ref[...] + vb_ref[...]
    pt = jax.nn.relu(lax.dot_general(nt, tw_ref[...], (((1,), (1,)), ((), ())),
                                     preferred_element_type=jnp.float32)
                     + tbias_ref[...])
    pv = jax.nn.relu(lax.dot_general(nv, vw_ref[...], (((1,), (1,)), ((), ())),
                                     preferred_element_type=jnp.float32)
                     + vbias_ref[...])
    out_ref[...] = (pt + pv) * 0.5


def _encoder(ft, fv, tg, tb, tw, tbias, vg, vb, vw, vbias):
    n = ft.shape[0]
    blk = 1000
    return pl.pallas_call(
        _enc_body,
        grid=(n // blk,),
        in_specs=[
            pl.BlockSpec((blk, ft.shape[1]), lambda i: (i, 0)),
            pl.BlockSpec((blk, fv.shape[1]), lambda i: (i, 0)),
            pl.BlockSpec((ft.shape[1],), lambda i: (0,)),
            pl.BlockSpec((ft.shape[1],), lambda i: (0,)),
            pl.BlockSpec((HID, ft.shape[1]), lambda i: (0, 0)),
            pl.BlockSpec((HID,), lambda i: (0,)),
            pl.BlockSpec((fv.shape[1],), lambda i: (0,)),
            pl.BlockSpec((fv.shape[1],), lambda i: (0,)),
            pl.BlockSpec((HID, fv.shape[1]), lambda i: (0, 0)),
            pl.BlockSpec((HID,), lambda i: (0,)),
        ],
        out_specs=pl.BlockSpec((blk, HID), lambda i: (i, 0)),
        out_shape=jax.ShapeDtypeStruct((n, HID), jnp.float32),
    )(ft, fv, tg, tb, tw, tbias, vg, vb, vw, vbias)


def _proj_body(h_ref, wl_ref, bl_ref, wr_ref, br_ref, xl_ref, xr_ref):
    hb = h_ref[...]
    for hd in range(H):
        wl = wl_ref[pl.ds(hd * HID, HID), :]
        wr = wr_ref[pl.ds(hd * HID, HID), :]
        xl_ref[hd] = lax.dot_general(hb, wl, (((1,), (1,)), ((), ())),
                                     preferred_element_type=jnp.float32) \
            + bl_ref[pl.ds(hd * HID, HID)]
        xr_ref[hd] = lax.dot_general(hb, wr, (((1,), (1,)), ((), ())),
                                     preferred_element_type=jnp.float32) \
            + br_ref[pl.ds(hd * HID, HID)]


def _project(h, Wl, bl, Wr, br):
    din = h.shape[1]
    blk = ROWS_PER_TILE
    out = jax.ShapeDtypeStruct((H, NP, HID), jnp.float32)
    return pl.pallas_call(
        _proj_body,
        grid=(NP // blk,),
        in_specs=[
            pl.BlockSpec((blk, din), lambda i: (i, 0)),
            pl.BlockSpec((H * HID, din), lambda i: (0, 0)),
            pl.BlockSpec((H * HID,), lambda i: (0,)),
            pl.BlockSpec((H * HID, din), lambda i: (0, 0)),
            pl.BlockSpec((H * HID,), lambda i: (0,)),
        ],
        out_specs=[
            pl.BlockSpec((H, blk, HID), lambda i: (0, i, 0)),
            pl.BlockSpec((H, blk, HID), lambda i: (0, i, 0)),
        ],
        out_shape=[out, out],
    )(h, Wl, bl, Wr, br)


def _combine_body(num_ref, den_ref, bias_ref, out_ref):
    for hd in range(H):
        n = num_ref[0, hd] + num_ref[1, hd]
        d = den_ref[0, hd, :, 0:1] + den_ref[1, hd, :, 0:1]
        out_ref[:, pl.ds(hd * HID, HID)] = (
            n / (d + 1e-16) + bias_ref[pl.ds(hd * HID, HID)])


def _combine(num, den16, bias):
    blk = ROWS_PER_TILE
    return pl.pallas_call(
        _combine_body,
        grid=(NP // blk,),
        in_specs=[
            pl.BlockSpec((2, H, blk, HID), lambda i: (0, 0, i, 0)),
            pl.BlockSpec((2, H, blk, 16), lambda i: (0, 0, i, 0)),
            pl.BlockSpec((H * HID,), lambda i: (0,)),
        ],
        out_specs=pl.BlockSpec((blk, H * HID), lambda i: (i, 0)),
        out_shape=jax.ShapeDtypeStruct((NP, H * HID), jnp.float32),
    )(num, den16, bias)


def _dec_body(g_ref, w1_ref, b1_ref, lg_ref, lb_ref, w2_ref, b2_ref, out_ref):
    z1 = lax.dot_general(g_ref[...], w1_ref[...], (((1,), (1,)), ((), ())),
                         preferred_element_type=jnp.float32) + b1_ref[...]
    m = jnp.mean(z1, axis=-1, keepdims=True)
    v = jnp.mean((z1 - m) ** 2, axis=-1, keepdims=True)
    z1 = jax.nn.relu((z1 - m) * lax.rsqrt(v + 1e-5) * lg_ref[...] + lb_ref[...])
    out_ref[...] = jnp.sum(z1 * w2_ref[...], axis=-1, keepdims=True) \
        + b2_ref[...]


def _decoder(glob, W1, b1, lg, lb, W2, b2):
    return pl.pallas_call(
        _dec_body,
        out_shape=jax.ShapeDtypeStruct((1, 1), jnp.float32),
    )(glob, W1, b1, lg, lb, W2, b2.reshape(1, 1))


def _edge_body(xl_ref, xr_ref, att_ref, srcg_ref, dstg_ref, dsts_ref,
               dstq_ref,
               num_out, den_out,
               num_s, den_s,
               A, B, D2, ig, jg, sc, dq, scv, scv2,
               attv, gsem1, gsem2, ssem1, ssem2):
    c = lax.axis_index("c")
    s = lax.axis_index("s")
    wid = c * 16 + s
    rows0 = s * ROWS_PER_TILE
    rows0d = s * DROWS_PER_TILE

    pltpu.sync_copy(att_ref, attv)

    zv = jnp.zeros((16,), jnp.float32)

    lanes = lax.broadcasted_iota(jnp.int32, (16,), 0)
    lane0 = jnp.where(lanes == 0, 1.0, 0.0)

    base_r = wid * CHUNKS

    def zrow(i, carry):
        for j in range(HID // 16):
            A[i, pl.ds(j * 16, 16)] = zv
            D2[i, pl.ds(j * 16, 16)] = zv
        return carry

    nz = (ROWS_PER_TILE + K - 1) // K
    zbases = [min(kk * K, ROWS_PER_TILE - K) for kk in range(nz)]
    nzd = (DROWS_PER_TILE + K - 1) // K
    zbases_d = [min(kk * K, DROWS_PER_TILE - K) for kk in range(nzd)]

    def _set_idx(ref, base):
        for q in range(K // 16):
            ref[pl.ds(q * 16, 16)] = lanes + (base + q * 16)

    for h in range(H):
        lax.fori_loop(0, K, zrow, 0)
        for zb in zbases:
            _set_idx(scv, rows0 + zb)
            pltpu.sync_copy(A, num_s.at[scv])
        for zb in zbases_d:
            _set_idx(scv2, rows0d + zb)
            pltpu.sync_copy(D2, den_s.at[scv2])
        plsc.subcore_barrier()

        def sup_body(si, carry):
            r0 = base_r + si * SUP
            pltpu.sync_copy(srcg_ref.at[h, pl.ds(r0, SUP)], ig)
            pltpu.sync_copy(dstg_ref.at[h, pl.ds(r0, SUP)], jg)
            pltpu.sync_copy(dsts_ref.at[pl.ds(r0, SUP)], sc)
            pltpu.sync_copy(dstq_ref.at[pl.ds(r0, SUP)], dq)
            for b in range(SUP):
                ga = pltpu.async_copy(xl_ref.at[ig.at[b]], A, gsem1)
                gb = pltpu.async_copy(xr_ref.at[jg.at[b]], B, gsem2)
                for q in range(K // 16):
                    scv[pl.ds(q * 16, 16)] = sc[b, pl.ds(q * 16, 16)]
                    scv2[pl.ds(q * 16, 16)] = dq[b, pl.ds(q * 16, 16)]
                ga.wait()
                gb.wait()

                def edge(e, ecarry):
                    acc = zv
                    for j in range(HID // 16):
                        a = A[e, pl.ds(j * 16, 16)]
                        r = B[e, pl.ds(j * 16, 16)]
                        t = a + r
                        m = jnp.maximum(t, t * 0.2)
                        acc = acc + m * attv[pl.ds(h * HID + j * 16, 16)]
                    ev = jnp.exp(jnp.broadcast_to(jnp.sum(acc), (16,)))
                    for j in range(HID // 16):
                        A[e, pl.ds(j * 16, 16)] = A[e, pl.ds(j * 16, 16)] * ev
                    dvec = plsc.load_gather(
                        scv, [jnp.broadcast_to(e, (16,))])
                    t_e = jnp.bitwise_and(dvec, 7)
                    sl0 = ev * lane0
                    for t in range(8):
                        D2[e, pl.ds(t * 16, 16)] = jnp.where(t_e == t, sl0, zv)
                    return ecarry

                lax.fori_loop(0, K, edge, 0)
                sa = pltpu.async_copy(A, num_s.at[scv], ssem1, add=True)
                sb = pltpu.async_copy(D2, den_s.at[scv2], ssem2, add=True)
                sa.wait()
                sb.wait()
            return carry

        lax.fori_loop(0, SUPS, sup_body, 0)
        plsc.subcore_barrier()
        for zb in zbases:
            _set_idx(scv, rows0 + zb)
            pltpu.sync_copy(num_s.at[scv], A)
            pltpu.sync_copy(A, num_out.at[c, h, pl.ds(rows0 + zb, K)])
        for zb in zbases_d:
            _set_idx(scv2, rows0d + zb)
            pltpu.sync_copy(den_s.at[scv2], D2)
            pltpu.sync_copy(D2, den_out.at[c, h, pl.ds(rows0d + zb, K)])
        plsc.subcore_barrier()


@functools.partial(
    pl.kernel,
    out_type=(jax.ShapeDtypeStruct((2, H, NP, HID), jnp.float32),
              jax.ShapeDtypeStruct((2, H, NPQ, HID), jnp.float32)),
    mesh=plsc.VectorSubcoreMesh(core_axis_name="c", subcore_axis_name="s"),
    compiler_params=pltpu.CompilerParams(needs_layout_passes=False),
    scratch_types=[
        pltpu.VMEM_SHARED((NP, HID), jnp.float32),
        pltpu.VMEM_SHARED((NPQ, HID), jnp.float32),
        pltpu.VMEM((K, HID), jnp.float32),
        pltpu.VMEM((K, HID), jnp.float32),
        pltpu.VMEM((K, HID), jnp.float32),
        pltpu.VMEM((SUP, K), jnp.int32),
        pltpu.VMEM((SUP, K), jnp.int32),
        pltpu.VMEM((SUP, K), jnp.int32),
        pltpu.VMEM((SUP, K), jnp.int32),
        pltpu.VMEM((K,), jnp.int32),
        pltpu.VMEM((K,), jnp.int32),
        pltpu.VMEM((H * HID,), jnp.float32),
        pltpu.SemaphoreType.DMA,
        pltpu.SemaphoreType.DMA,
        pltpu.SemaphoreType.DMA,
        pltpu.SemaphoreType.DMA,
    ],
)
def _edge_stage(xl_ref, xr_ref, att_ref, srcg_ref, dstg_ref, dsts_ref,
                dstq_ref, num_out, den_out, *scratch):
    _edge_body(xl_ref, xr_ref, att_ref, srcg_ref, dstg_ref, dsts_ref,
               dstq_ref, num_out, den_out, *scratch)


def _gat_layer(h, src_g, dst_g, dst_s, dst_q, Wl, bl, Wr, br, att, bias):
    xl, xr = _project(h, Wl, bl, Wr, br)
    num, den = _edge_stage(xl.reshape(H * NP, HID), xr.reshape(H * NP, HID),
                           att.reshape(H * HID), src_g, dst_g, dst_s, dst_q)
    den_nodes = den.reshape(2, H, NPQ, 8, 16)[..., 0].reshape(
        2, H, NPQ * 8)[:, :, :NP]
    den16 = jnp.broadcast_to(den_nodes[..., None], (2, H, NP, 16))
    return _combine(num, den16, bias)


def kernel(feat_text, feat_video, x_hie_conv, x_hie_speaker, edge_index, speaker_index, global_index, text_ln_g, text_ln_b, text_W, text_b, video_ln_g, video_ln_b, video_W, video_b, tok_hc, tok_sp, g1_Wl, g1_bl, g1_Wr, g1_br, g1_att, g1_bias, g2_Wl, g2_bl, g2_Wr, g2_br, g2_att, g2_bias, g3_Wl, g3_bl, g3_Wr, g3_br, g3_att, g3_bias, dec_W1, dec_b1, dec_ln_g, dec_ln_b, dec_W2, dec_b2):
    x = _encoder(feat_text, feat_video, text_ln_g, text_ln_b, text_W, text_b,
                 video_ln_g, video_ln_b, video_W, video_b)
    hc = jnp.tile(tok_hc[None, :], (x_hie_conv.shape[0], 1))
    sp = jnp.tile(tok_sp[None, :], (x_hie_speaker.shape[0], 1))
    h = jnp.concatenate(
        [x, hc, sp, jnp.zeros((NP - N_TOT, HID), jnp.float32)], axis=0)

    loops = jnp.arange(N_TOT, dtype=jnp.int32)
    pad = EP - E_RAW
    src = jnp.concatenate([edge_index[0].astype(jnp.int32), loops,
                           jnp.zeros((pad,), jnp.int32)])
    dst = jnp.concatenate([edge_index[1].astype(jnp.int32), loops,
                           jnp.full((pad,), N_TOT, jnp.int32)])
    order = jnp.argsort(dst)
    src = src[order].reshape(K, EROWS).T.reshape(EP)
    dst = dst[order].reshape(K, EROWS).T.reshape(EP)

    offs = (jnp.arange(H, dtype=jnp.int32) * NP)[:, None]
    src_g = (src[None, :] + offs).reshape(H, EROWS, K)
    dst_g = (dst[None, :] + offs).reshape(H, EROWS, K)
    dst_q = (dst >> 3).reshape(EROWS, K)
    dst = dst.reshape(EROWS, K)

    h = _gat_layer(h, src_g, dst_g, dst, dst_q, g1_Wl, g1_bl, g1_Wr, g1_br,
                   g1_att, g1_bias)
    h = _gat_layer(h, src_g, dst_g, dst, dst_q, g2_Wl, g2_bl, g2_Wr, g2_br,
                   g2_att, g2_bias)
    h = _gat_layer(h, src_g, dst_g, dst, dst_q, g3_Wl, g3_bl, g3_Wr, g3_br,
                   g3_att, g3_bias)

    glob = lax.dynamic_slice(h, (global_index, 0), (1, H * HID))
    return _decoder(glob, dec_W1, dec_b1, dec_ln_g, dec_ln_b, dec_W2, dec_b2)


# speedup vs baseline: 19.0875x; 1.5541x over previous
import functools

import jax
import jax.numpy as jnp
from jax import lax
from jax.experimental import pallas as pl
from jax.experimental.pallas import tpu as pltpu
from jax.experimental.pallas import tpu_sc as plsc

H = 4
HID = 128
N_CONV = 10000
N_TOT = 10600
NP = 10624
ROWS_PER_TILE = NP // 16
E_RAW = 169600 + N_TOT
EP = 180224
E_PER_TILE = EP // 32
K = 64
SUP = 8
CHUNKS = E_PER_TILE // K
SUPS = CHUNKS // SUP
EROWS = EP // K
NPQ = 1408
DROWS_PER_TILE = NPQ // 16


def _enc_body(ft_ref, fv_ref, tg_ref, tb_ref, tw_ref, tbias_ref,
              vg_ref, vb_ref, vw_ref, vbias_ref, out_ref):
    ft = ft_ref[...]
    fv = fv_ref[...]
    m = jnp.mean(ft, axis=-1, keepdims=True)
    v = jnp.mean((ft - m) ** 2, axis=-1, keepdims=True)
    nt = (ft - m) * lax.rsqrt(v + 1e-5) * tg_ref[...] + tb_ref[...]
    m2 = jnp.mean(fv, axis=-1, keepdims=True)
    v2 = jnp.mean((fv - m2) ** 2, axis=-1, keepdims=True)
    nv = (fv - m2) * lax.rsqrt(v2 + 1e-5) * vg_ref[...] + vb_ref[...]
    pt = jax.nn.relu(lax.dot_general(nt, tw_ref[...], (((1,), (1,)), ((), ())),
                                     preferred_element_type=jnp.float32)
                     + tbias_ref[...])
    pv = jax.nn.relu(lax.dot_general(nv, vw_ref[...], (((1,), (1,)), ((), ())),
                                     preferred_element_type=jnp.float32)
                     + vbias_ref[...])
    out_ref[...] = (pt + pv) * 0.5


def _encoder(ft, fv, tg, tb, tw, tbias, vg, vb, vw, vbias):
    n = ft.shape[0]
    blk = 1000
    return pl.pallas_call(
        _enc_body,
        grid=(n // blk,),
        in_specs=[
            pl.BlockSpec((blk, ft.shape[1]), lambda i: (i, 0)),
            pl.BlockSpec((blk, fv.shape[1]), lambda i: (i, 0)),
            pl.BlockSpec((ft.shape[1],), lambda i: (0,)),
            pl.BlockSpec((ft.shape[1],), lambda i: (0,)),
            pl.BlockSpec((HID, ft.shape[1]), lambda i: (0, 0)),
            pl.BlockSpec((HID,), lambda i: (0,)),
            pl.BlockSpec((fv.shape[1],), lambda i: (0,)),
            pl.BlockSpec((fv.shape[1],), lambda i: (0,)),
            pl.BlockSpec((HID, fv.shape[1]), lambda i: (0, 0)),
            pl.BlockSpec((HID,), lambda i: (0,)),
        ],
        out_specs=pl.BlockSpec((blk, HID), lambda i: (i, 0)),
        out_shape=jax.ShapeDtypeStruct((n, HID), jnp.float32),
    )(ft, fv, tg, tb, tw, tbias, vg, vb, vw, vbias)


def _proj_body(h_ref, wl_ref, bl_ref, wr_ref, br_ref, xl_ref, xr_ref):
    hb = h_ref[...]
    for hd in range(H):
        wl = wl_ref[pl.ds(hd * HID, HID), :]
        wr = wr_ref[pl.ds(hd * HID, HID), :]
        xl_ref[hd] = lax.dot_general(hb, wl, (((1,), (1,)), ((), ())),
                                     preferred_element_type=jnp.float32) \
            + bl_ref[pl.ds(hd * HID, HID)]
        xr_ref[hd] = lax.dot_general(hb, wr, (((1,), (1,)), ((), ())),
                                     preferred_element_type=jnp.float32) \
            + br_ref[pl.ds(hd * HID, HID)]


def _project(h, Wl, bl, Wr, br):
    din = h.shape[1]
    blk = ROWS_PER_TILE
    out = jax.ShapeDtypeStruct((H, NP, HID), jnp.float32)
    return pl.pallas_call(
        _proj_body,
        grid=(NP // blk,),
        in_specs=[
            pl.BlockSpec((blk, din), lambda i: (i, 0)),
            pl.BlockSpec((H * HID, din), lambda i: (0, 0)),
            pl.BlockSpec((H * HID,), lambda i: (0,)),
            pl.BlockSpec((H * HID, din), lambda i: (0, 0)),
            pl.BlockSpec((H * HID,), lambda i: (0,)),
        ],
        out_specs=[
            pl.BlockSpec((H, blk, HID), lambda i: (0, i, 0)),
            pl.BlockSpec((H, blk, HID), lambda i: (0, i, 0)),
        ],
        out_shape=[out, out],
    )(h, Wl, bl, Wr, br)


def _combine_body(num_ref, den_ref, bias_ref, out_ref):
    for hd in range(H):
        n = num_ref[0, hd] + num_ref[1, hd]
        d = den_ref[0, hd, :, 0:1] + den_ref[1, hd, :, 0:1]
        out_ref[:, pl.ds(hd * HID, HID)] = (
            n / (d + 1e-16) + bias_ref[pl.ds(hd * HID, HID)])


def _combine(num, den16, bias):
    blk = ROWS_PER_TILE
    return pl.pallas_call(
        _combine_body,
        grid=(NP // blk,),
        in_specs=[
            pl.BlockSpec((2, H, blk, HID), lambda i: (0, 0, i, 0)),
            pl.BlockSpec((2, H, blk, 16), lambda i: (0, 0, i, 0)),
            pl.BlockSpec((H * HID,), lambda i: (0,)),
        ],
        out_specs=pl.BlockSpec((blk, H * HID), lambda i: (i, 0)),
        out_shape=jax.ShapeDtypeStruct((NP, H * HID), jnp.float32),
    )(num, den16, bias)


def _dec_body(g_ref, w1_ref, b1_ref, lg_ref, lb_ref, w2_ref, b2_ref, out_ref):
    z1 = lax.dot_general(g_ref[...], w1_ref[...], (((1,), (1,)), ((), ())),
                         preferred_element_type=jnp.float32) + b1_ref[...]
    m = jnp.mean(z1, axis=-1, keepdims=True)
    v = jnp.mean((z1 - m) ** 2, axis=-1, keepdims=True)
    z1 = jax.nn.relu((z1 - m) * lax.rsqrt(v + 1e-5) * lg_ref[...] + lb_ref[...])
    out_ref[...] = jnp.sum(z1 * w2_ref[...], axis=-1, keepdims=True) \
        + b2_ref[...]


def _decoder(glob, W1, b1, lg, lb, W2, b2):
    return pl.pallas_call(
        _dec_body,
        out_shape=jax.ShapeDtypeStruct((1, 1), jnp.float32),
    )(glob, W1, b1, lg, lb, W2, b2.reshape(1, 1))


def _edge_body(xl_ref, xr_ref, att_ref, srcg_ref, dstg_ref, dsts_ref,
               dstq_ref,
               num_out, den_out,
               num_s, den_s,
               A, B, D2, ig, jg, sc, dq, scv, scv2,
               attv, gsem1, gsem2, ssem1, ssem2):
    c = lax.axis_index("c")
    s = lax.axis_index("s")
    wid = c * 16 + s
    rows0 = s * ROWS_PER_TILE
    rows0d = s * DROWS_PER_TILE

    pltpu.sync_copy(att_ref, attv)

    zv = jnp.zeros((16,), jnp.float32)

    lanes = lax.broadcasted_iota(jnp.int32, (16,), 0)
    lane0 = jnp.where(lanes == 0, 1.0, 0.0)

    base_r = wid * CHUNKS

    def zrow(i, carry):
        for j in range(HID // 16):
            A[i, pl.ds(j * 16, 16)] = zv
            D2[i, pl.ds(j * 16, 16)] = zv
        return carry

    nz = (ROWS_PER_TILE + K - 1) // K
    zbases = [min(kk * K, ROWS_PER_TILE - K) for kk in range(nz)]
    nzd = (DROWS_PER_TILE + K - 1) // K
    zbases_d = [min(kk * K, DROWS_PER_TILE - K) for kk in range(nzd)]

    def _set_idx(ref, base):
        for q in range(K // 16):
            ref[pl.ds(q * 16, 16)] = lanes + (base + q * 16)

    def head_body(h, hcarry):
        att_h = [attv[pl.ds(h * HID + j * 16, 16)] for j in range(HID // 16)]
        lax.fori_loop(0, K, zrow, 0)
        for zb in zbases:
            _set_idx(scv, rows0 + zb)
            pltpu.sync_copy(A, num_s.at[scv])
        for zb in zbases_d:
            _set_idx(scv2, rows0d + zb)
            pltpu.sync_copy(D2, den_s.at[scv2])
        plsc.subcore_barrier()

        def sup_body(si, carry):
            r0 = base_r + si * SUP
            pltpu.sync_copy(srcg_ref.at[h, pl.ds(r0, SUP)], ig)
            pltpu.sync_copy(dstg_ref.at[h, pl.ds(r0, SUP)], jg)
            pltpu.sync_copy(dsts_ref.at[pl.ds(r0, SUP)], sc)
            pltpu.sync_copy(dstq_ref.at[pl.ds(r0, SUP)], dq)
            for b in range(SUP):
                ga = pltpu.async_copy(xl_ref.at[ig.at[b]], A, gsem1)
                gb = pltpu.async_copy(xr_ref.at[jg.at[b]], B, gsem2)
                for q in range(K // 16):
                    scv[pl.ds(q * 16, 16)] = sc[b, pl.ds(q * 16, 16)]
                    scv2[pl.ds(q * 16, 16)] = dq[b, pl.ds(q * 16, 16)]
                ga.wait()
                gb.wait()

                @plsc.parallel_loop(0, K, unroll=2)
                def _edge(e):
                    acc = zv
                    for j in range(HID // 16):
                        a = A[e, pl.ds(j * 16, 16)]
                        r = B[e, pl.ds(j * 16, 16)]
                        t = a + r
                        m = jnp.maximum(t, t * 0.2)
                        acc = acc + m * att_h[j]
                    ev = jnp.exp(jnp.broadcast_to(jnp.sum(acc), (16,)))
                    for j in range(HID // 16):
                        A[e, pl.ds(j * 16, 16)] = A[e, pl.ds(j * 16, 16)] * ev
                    dvec = plsc.load_gather(
                        scv, [jnp.broadcast_to(e, (16,))])
                    t_e = jnp.bitwise_and(dvec, 7)
                    sl0 = ev * lane0
                    for t in range(8):
                        D2[e, pl.ds(t * 16, 16)] = jnp.where(t_e == t, sl0, zv)
                sa = pltpu.async_copy(A, num_s.at[scv], ssem1, add=True)
                sb = pltpu.async_copy(D2, den_s.at[scv2], ssem2, add=True)
                sa.wait()
                sb.wait()
            return carry

        lax.fori_loop(0, SUPS, sup_body, 0)
        plsc.subcore_barrier()
        for zb in zbases:
            _set_idx(scv, rows0 + zb)
            pltpu.sync_copy(num_s.at[scv], A)
            pltpu.sync_copy(A, num_out.at[c, h, pl.ds(rows0 + zb, K)])
        for zb in zbases_d:
            _set_idx(scv2, rows0d + zb)
            pltpu.sync_copy(den_s.at[scv2], D2)
            pltpu.sync_copy(D2, den_out.at[c, h, pl.ds(rows0d + zb, K)])
        plsc.subcore_barrier()
        return hcarry

    lax.fori_loop(0, H, head_body, 0)


@functools.partial(
    pl.kernel,
    out_type=(jax.ShapeDtypeStruct((2, H, NP, HID), jnp.float32),
              jax.ShapeDtypeStruct((2, H, NPQ, HID), jnp.float32)),
    mesh=plsc.VectorSubcoreMesh(core_axis_name="c", subcore_axis_name="s"),
    compiler_params=pltpu.CompilerParams(needs_layout_passes=False),
    scratch_types=[
        pltpu.VMEM_SHARED((NP, HID), jnp.float32),
        pltpu.VMEM_SHARED((NPQ, HID), jnp.float32),
        pltpu.VMEM((K, HID), jnp.float32),
        pltpu.VMEM((K, HID), jnp.float32),
        pltpu.VMEM((K, HID), jnp.float32),
        pltpu.VMEM((SUP, K), jnp.int32),
        pltpu.VMEM((SUP, K), jnp.int32),
        pltpu.VMEM((SUP, K), jnp.int32),
        pltpu.VMEM((SUP, K), jnp.int32),
        pltpu.VMEM((K,), jnp.int32),
        pltpu.VMEM((K,), jnp.int32),
        pltpu.VMEM((H * HID,), jnp.float32),
        pltpu.SemaphoreType.DMA,
        pltpu.SemaphoreType.DMA,
        pltpu.SemaphoreType.DMA,
        pltpu.SemaphoreType.DMA,
    ],
)
def _edge_stage(xl_ref, xr_ref, att_ref, srcg_ref, dstg_ref, dsts_ref,
                dstq_ref, num_out, den_out, *scratch):
    _edge_body(xl_ref, xr_ref, att_ref, srcg_ref, dstg_ref, dsts_ref,
               dstq_ref, num_out, den_out, *scratch)


def _gat_layer(h, src_g, dst_g, dst_s, dst_q, Wl, bl, Wr, br, att, bias):
    xl, xr = _project(h, Wl, bl, Wr, br)
    num, den = _edge_stage(xl.reshape(H * NP, HID), xr.reshape(H * NP, HID),
                           att.reshape(H * HID), src_g, dst_g, dst_s, dst_q)
    den_nodes = den.reshape(2, H, NPQ, 8, 16)[..., 0].reshape(
        2, H, NPQ * 8)[:, :, :NP]
    den16 = jnp.broadcast_to(den_nodes[..., None], (2, H, NP, 16))
    return _combine(num, den16, bias)


def kernel(feat_text, feat_video, x_hie_conv, x_hie_speaker, edge_index, speaker_index, global_index, text_ln_g, text_ln_b, text_W, text_b, video_ln_g, video_ln_b, video_W, video_b, tok_hc, tok_sp, g1_Wl, g1_bl, g1_Wr, g1_br, g1_att, g1_bias, g2_Wl, g2_bl, g2_Wr, g2_br, g2_att, g2_bias, g3_Wl, g3_bl, g3_Wr, g3_br, g3_att, g3_bias, dec_W1, dec_b1, dec_ln_g, dec_ln_b, dec_W2, dec_b2):
    x = _encoder(feat_text, feat_video, text_ln_g, text_ln_b, text_W, text_b,
                 video_ln_g, video_ln_b, video_W, video_b)
    hc = jnp.tile(tok_hc[None, :], (x_hie_conv.shape[0], 1))
    sp = jnp.tile(tok_sp[None, :], (x_hie_speaker.shape[0], 1))
    h = jnp.concatenate(
        [x, hc, sp, jnp.zeros((NP - N_TOT, HID), jnp.float32)], axis=0)

    loops = jnp.arange(N_TOT, dtype=jnp.int32)
    pad = EP - E_RAW
    src = jnp.concatenate([edge_index[0].astype(jnp.int32), loops,
                           jnp.zeros((pad,), jnp.int32)])
    dst = jnp.concatenate([edge_index[1].astype(jnp.int32), loops,
                           jnp.full((pad,), N_TOT, jnp.int32)])
    order = jnp.argsort(dst)
    src = src[order].reshape(K, EROWS).T.reshape(EP)
    dst = dst[order].reshape(K, EROWS).T.reshape(EP)

    offs = (jnp.arange(H, dtype=jnp.int32) * NP)[:, None]
    src_g = (src[None, :] + offs).reshape(H, EROWS, K)
    dst_g = (dst[None, :] + offs).reshape(H, EROWS, K)
    dst_q = (dst >> 3).reshape(EROWS, K)
    dst = dst.reshape(EROWS, K)

    h = _gat_layer(h, src_g, dst_g, dst, dst_q, g1_Wl, g1_bl, g1_Wr, g1_br,
                   g1_att, g1_bias)
    h = _gat_layer(h, src_g, dst_g, dst, dst_q, g2_Wl, g2_bl, g2_Wr, g2_br,
                   g2_att, g2_bias)
    h = _gat_layer(h, src_g, dst_g, dst, dst_q, g3_Wl, g3_bl, g3_Wr, g3_br,
                   g3_att, g3_bias)

    glob = lax.dynamic_slice(h, (global_index, 0), (1, H * HID))
    return _decoder(glob, dec_W1, dec_b1, dec_ln_g, dec_ln_b, dec_W2, dec_b2)
